# Initial kernel scaffold; baseline (speedup 1.0000x reference)
import jax
import jax.numpy as jnp
from jax.experimental import pallas as pl


def kernel(inputs):
    raise NotImplementedError("write your pallas kernel here")


# speedup vs baseline: 1.8869x; 1.8869x over previous
import functools
import math

import jax
import jax.numpy as jnp
import numpy as np
from jax import lax
from jax.experimental import pallas as pl
from jax.experimental.pallas import tpu as pltpu
from jax.experimental.pallas import tpu_sc as plsc

R = 1536
C = 16384
T = R // 3
NC, NS = 2, 16
NW = NC * NS
GPW = T // NW
CH = C // 2
NCHUNK = GPW * 2
NBUF = 4
LANES = 16
PI = float(np.pi)
TWO_PI = float(2.0 * np.pi)

def _rot_value() -> float:
    def rotl(x, r):
        return np.uint32((int(x) << r | int(x) >> (32 - r)) & 0xFFFFFFFF)

    k0, k1 = np.uint32(0), np.uint32(42)
    ks = [k0, k1, np.uint32(int(k0) ^ int(k1) ^ 0x1BD11BDA)]
    x0, x1 = np.uint32(int(ks[0])), np.uint32(int(ks[1]))
    rotations = [(13, 15, 26, 6), (17, 29, 16, 24)]
    for i in range(5):
        for r in rotations[i % 2]:
            x0 = np.uint32((int(x0) + int(x1)) & 0xFFFFFFFF)
            x1 = rotl(x1, r)
            x1 = np.uint32(int(x1) ^ int(x0))
        x0 = np.uint32((int(x0) + int(ks[(i + 1) % 3])) & 0xFFFFFFFF)
        x1 = np.uint32((int(x1) + int(ks[(i + 2) % 3]) + i + 1) & 0xFFFFFFFF)
    bits = np.uint32(int(x0) ^ int(x1))
    mantissa = np.uint32((int(bits) >> 9) | 0x3F800000)
    u = mantissa.view(np.float32) - np.float32(1.0)
    lo, hi = np.float32(-np.pi), np.float32(np.pi)
    val = np.float32(math.fma(float(u), float(hi - lo), float(lo)))
    return float(np.maximum(lo, val))


def _make_sc_kernel(rot: float):
    mesh = plsc.VectorSubcoreMesh(core_axis_name="c", subcore_axis_name="s",
                                  num_cores=NC, num_subcores=NS)

    @functools.partial(
        pl.kernel,
        out_type=jax.ShapeDtypeStruct((R, C), jnp.float32),
        mesh=mesh,
        scratch_types=(
            [pltpu.VMEM((NBUF, 3, CH), jnp.float32)]
            + [pltpu.SemaphoreType.DMA] * NBUF
            + [pltpu.SemaphoreType.DMA] * NBUF
        ),
        compiler_params=pltpu.CompilerParams(use_tc_tiling_on_sc=False),
    )
    def phi_rotate(in_hbm, out_hbm, buf, *sems):
        isems = sems[:NBUF]
        osems = sems[NBUF:]
        wid = lax.axis_index("c") * NS + lax.axis_index("s")
        row0 = wid * (GPW * 3)

        def start_in(i, s):
            g, h = divmod(i, 2)
            return pltpu.async_copy(
                in_hbm.at[pl.ds(row0 + 3 * g, 3), pl.ds(h * CH, CH)],
                buf.at[s], isems[s])

        def start_out(i, s):
            g, h = divmod(i, 2)
            return pltpu.async_copy(
                buf.at[s],
                out_hbm.at[pl.ds(row0 + 3 * g, 3), pl.ds(h * CH, CH)],
                osems[s])

        def compute(s):
            def body(j, carry):
                sl = pl.ds(j * LANES, LANES)
                pt = buf[s, 0, sl]
                ph = buf[s, 2, sl]
                ph = ph + jnp.where(pt != 0.0, rot, 0.0)
                ph = jnp.where(ph > PI, ph - TWO_PI, ph)
                ph = jnp.where(ph < -PI, ph + TWO_PI, ph)
                buf[s, 2, sl] = ph
                return carry
            lax.fori_loop(0, CH // LANES, body, 0, unroll=8)

        cin = [None] * NBUF
        cout = [None] * NBUF
        for j in range(NBUF - 1):
            cin[j] = start_in(j, j)
        for i in range(NCHUNK):
            s = i % NBUF
            cin[s].wait()
            compute(s)
            cout[s] = start_out(i, s)
            nxt = i + (NBUF - 1)
            if nxt < NCHUNK:
                ns = nxt % NBUF
                if cout[ns] is not None:
                    cout[ns].wait()
                cin[ns] = start_in(nxt, ns)
        for i in range(NCHUNK - NBUF, NCHUNK):
            cout[i % NBUF].wait()

    return phi_rotate


def kernel(inputs):
    return _make_sc_kernel(_rot_value())(inputs)
